# baseline (speedup 1.0000x reference)
import functools

import jax
import jax.numpy as jnp
from jax import lax
from jax.experimental import pallas as pl
from jax.experimental.pallas import tpu as pltpu
from jax.experimental.pallas import tpu_sc as plsc

NUM_NODES = 1000000
EMBED_DIM = 64

_NBATCH = 16384
_S = 50
_SP = 56
_DP = 128
_NW = 32
_BPW = _NBATCH // _NW
_SB = 4
_NB = 4
_T = _BPW // (_SB * _NB)


def _make_gather():
    mesh = plsc.VectorSubcoreMesh(core_axis_name="c", subcore_axis_name="s")

    @functools.partial(
        pl.kernel,
        out_type=jax.ShapeDtypeStruct((_NBATCH, _SP, _DP), jnp.float32),
        mesh=mesh,
        scratch_types=(
            [pltpu.VMEM((_SB * _SP,), jnp.int32)] * _NB
            + [pltpu.VMEM((_SB, _SP, _DP), jnp.float32)] * _NB
            + [pltpu.SemaphoreType.DMA] * (3 * _NB)
        ),
        compiler_params=pltpu.CompilerParams(use_tc_tiling_on_sc=True),
    )
    def gather_kernel(idx_hbm, table_hbm, out_hbm, *refs):
        idxb = refs[:_NB]
        rows = refs[_NB:2 * _NB]
        isem = refs[2 * _NB:3 * _NB]
        gsem = refs[3 * _NB:4 * _NB]
        ssem = refs[4 * _NB:]

        wid = lax.axis_index("s") * 2 + lax.axis_index("c")
        b_base = wid * _BPW

        def issue_idx(blk, b):
            pltpu.async_copy(
                idx_hbm.at[pl.ds((b_base + blk * _SB) * _SP, _SB * _SP)],
                idxb[b], isem[b])

        def issue_gathers(b):
            for k in range(_SB):
                pltpu.async_copy(
                    table_hbm.at[idxb[b].at[pl.ds(k * _SP, _SP)]],
                    rows[b].at[k],
                    gsem[b])

        for b in range(_NB):
            issue_idx(b, b)

        def body(t, carry):
            for b in range(_NB):
                pltpu.make_async_copy(
                    idx_hbm.at[pl.ds(0, _SB * _SP)], idxb[b], isem[b]).wait()
                pl.when(t != 0)(functools.partial(
                    lambda b: pltpu.make_async_copy(
                        rows[b], out_hbm.at[pl.ds(0, _SB)], ssem[b]).wait(),
                    b))
                issue_gathers(b)
            for b in range(_NB):
                blk = t * _NB + b
                for k in range(_SB):
                    pltpu.make_async_copy(
                        table_hbm.at[idxb[b].at[pl.ds(k * _SP, _SP)]],
                        rows[b].at[k],
                        gsem[b]).wait()
                pltpu.async_copy(
                    rows[b],
                    out_hbm.at[pl.ds(b_base + blk * _SB, _SB)],
                    ssem[b])
                pl.when(t != _T - 1)(
                    functools.partial(issue_idx, (t + 1) * _NB + b, b))
            return carry

        lax.fori_loop(0, _T, body, 0, unroll=False)

        for b in range(_NB):
            pltpu.make_async_copy(
                rows[b], out_hbm.at[pl.ds(0, _SB)], ssem[b]).wait()

    return gather_kernel


_gather = _make_gather()


def kernel(indices, weight):
    idx = indices.astype(jnp.int32)
    idxp = jnp.pad(idx, ((0, 0), (0, _SP - _S)))
    w2 = jnp.reshape(weight, (NUM_NODES // 2, _DP))
    outp = _gather((idxp >> 1).reshape(-1), w2)
    odd = (idx & 1)[:, :, None] == 1
    return jnp.where(
        odd, outp[:, :_S, EMBED_DIM:], outp[:, :_S, :EMBED_DIM])


# speedup vs baseline: 1.8891x; 1.8891x over previous
import functools

import jax
import jax.numpy as jnp
from jax import lax
from jax.experimental import pallas as pl
from jax.experimental.pallas import tpu as pltpu
from jax.experimental.pallas import tpu_sc as plsc

NUM_NODES = 1000000
EMBED_DIM = 64

_NBATCH = 16384
_S = 50
_SP = 56
_DP = 128
_NW = 32
_BPW = _NBATCH // _NW
_SB = 4
_NB = 4
_T = _BPW // (_SB * _NB)


def _make_gather():
    mesh = plsc.VectorSubcoreMesh(core_axis_name="c", subcore_axis_name="s")

    @functools.partial(
        pl.kernel,
        out_type=jax.ShapeDtypeStruct((_NBATCH, _SP, _DP), jnp.float32),
        mesh=mesh,
        scratch_types=(
            [pltpu.VMEM((_SB * _SP,), jnp.int32)] * _NB
            + [pltpu.VMEM((_SB, _SP, EMBED_DIM), jnp.float32)] * _NB
            + [pltpu.SemaphoreType.DMA] * (3 * _NB)
        ),
        compiler_params=pltpu.CompilerParams(use_tc_tiling_on_sc=False),
    )
    def gather_kernel(idx_hbm, table_hbm, out_hbm, *refs):
        idxb = refs[:_NB]
        rows = refs[_NB:2 * _NB]
        isem = refs[2 * _NB:3 * _NB]
        gsem = refs[3 * _NB:4 * _NB]
        ssem = refs[4 * _NB:]

        wid = lax.axis_index("s") * 2 + lax.axis_index("c")
        b_base = wid * _BPW

        def issue_idx(blk, b):
            pltpu.async_copy(
                idx_hbm.at[pl.ds((b_base + blk * _SB) * _SP, _SB * _SP)],
                idxb[b], isem[b])

        def issue_gathers(b):
            for k in range(_SB):
                pltpu.async_copy(
                    table_hbm.at[idxb[b].at[pl.ds(k * _SP, _SP)]],
                    rows[b].at[k],
                    gsem[b])

        for b in range(_NB):
            issue_idx(b, b)

        def body(t, carry):
            for b in range(_NB):
                pltpu.make_async_copy(
                    idx_hbm.at[pl.ds(0, _SB * _SP)], idxb[b], isem[b]).wait()
                pl.when(t != 0)(functools.partial(
                    lambda b: pltpu.make_async_copy(
                        rows[b],
                        out_hbm.at[pl.ds(0, _SB), :, pl.ds(0, EMBED_DIM)],
                        ssem[b]).wait(),
                    b))
                issue_gathers(b)
            for b in range(_NB):
                blk = t * _NB + b
                for k in range(_SB):
                    pltpu.make_async_copy(
                        table_hbm.at[idxb[b].at[pl.ds(k * _SP, _SP)]],
                        rows[b].at[k],
                        gsem[b]).wait()
                pltpu.async_copy(
                    rows[b],
                    out_hbm.at[pl.ds(b_base + blk * _SB, _SB), :,
                               pl.ds(0, EMBED_DIM)],
                    ssem[b])
                pl.when(t != _T - 1)(
                    functools.partial(issue_idx, (t + 1) * _NB + b, b))
            return carry

        lax.fori_loop(0, _T, body, 0, unroll=False)

        for b in range(_NB):
            pltpu.make_async_copy(
                rows[b], out_hbm.at[pl.ds(0, _SB), :, pl.ds(0, EMBED_DIM)],
                ssem[b]).wait()

    return gather_kernel


_gather = _make_gather()


def kernel(indices, weight):
    idxp = jnp.pad(indices.astype(jnp.int32), ((0, 0), (0, _SP - _S)))
    outp = _gather(idxp.reshape(-1), weight)
    return outp[:, :_S, :EMBED_DIM]


# speedup vs baseline: 4.6922x; 2.4838x over previous
import functools

import jax
import jax.numpy as jnp
from jax import lax
from jax.experimental import pallas as pl
from jax.experimental.pallas import tpu as pltpu
from jax.experimental.pallas import tpu_sc as plsc

NUM_NODES = 1000000
EMBED_DIM = 64

_NBATCH = 16384
_S = 50
_SP = 56
_NW = 32
_BPW = _NBATCH // _NW
_SB = 4
_NB = 4
_T = _BPW // (_SB * _NB)


def _make_gather():
    mesh = plsc.VectorSubcoreMesh(core_axis_name="c", subcore_axis_name="s")

    @functools.partial(
        pl.kernel,
        out_type=jax.ShapeDtypeStruct((_NBATCH, _S, EMBED_DIM), jnp.float32),
        mesh=mesh,
        scratch_types=(
            [pltpu.VMEM((_BPW * _SP,), jnp.int32)]
            + [pltpu.VMEM((_SB, _S, EMBED_DIM), jnp.float32)] * _NB
            + [pltpu.SemaphoreType.DMA] * (2 * _NB)
        ),
        compiler_params=pltpu.CompilerParams(use_tc_tiling_on_sc=False),
    )
    def gather_kernel(idx_hbm, table_hbm, out_hbm, idx_v, *bufs_and_sems):
        bufs = bufs_and_sems[:_NB]
        gsem = bufs_and_sems[_NB:2 * _NB]
        ssem = bufs_and_sems[2 * _NB:]

        wid = lax.axis_index("s") * 2 + lax.axis_index("c")
        b_base = wid * _BPW
        pltpu.sync_copy(idx_hbm.at[pl.ds(b_base * _SP, _BPW * _SP)], idx_v)

        def issue_gathers(blk, b):
            for k in range(_SB):
                pltpu.async_copy(
                    table_hbm.at[idx_v.at[pl.ds((blk * _SB + k) * _SP, _S)]],
                    bufs[b].at[k],
                    gsem[b])

        for b in range(_NB):
            issue_gathers(b, b)

        def body(t, carry):
            for b in range(_NB):
                blk = t * _NB + b
                for k in range(_SB):
                    pltpu.make_async_copy(
                        table_hbm.at[idx_v.at[pl.ds(0, _S)]],
                        bufs[b].at[k],
                        gsem[b]).wait()
                pltpu.async_copy(
                    bufs[b],
                    out_hbm.at[pl.ds(b_base + blk * _SB, _SB)],
                    ssem[b])
            for b in range(_NB):
                pltpu.make_async_copy(
                    bufs[b], out_hbm.at[pl.ds(0, _SB)], ssem[b]).wait()
                pl.when(t != _T - 1)(
                    functools.partial(issue_gathers, (t + 1) * _NB + b, b))
            return carry

        lax.fori_loop(0, _T, body, 0, unroll=False)

    return gather_kernel


_gather = _make_gather()


def kernel(indices, weight):
    idxp = jnp.pad(indices.astype(jnp.int32), ((0, 0), (0, _SP - _S)))
    return _gather(idxp.reshape(-1), weight)
